# baseline (speedup 1.0000x reference)
import functools

import jax
import jax.numpy as jnp
from jax import lax
from jax.experimental import pallas as pl
from jax.experimental.pallas import tpu as pltpu
from jax.experimental.pallas import tpu_sc as plsc

N = 10000
E = 320000
D = 128
ED = 16
H = 128
G = 64

NC = 2
NS = 16
NW = NC * NS
CHUNK = 128
CH = 80
E_PAD = NW * CH * CHUNK
N_PAD = 10240
ROWS_PER_TILE = N_PAD // NS

RN = 2000
RE = 4096

_f32 = jnp.float32
_bf16 = jnp.bfloat16


def _full(shape):
    return pl.BlockSpec(shape, lambda i: (0,) * len(shape))


def _rows(shape):
    return pl.BlockSpec(shape, lambda i: (i,) + (0,) * (len(shape) - 1))


def _prologue_body(x_ref, wp_ref, bp_ref, w1a_ref, b1_ref, h_ref, a_ref):
    h = jnp.dot(x_ref[...], wp_ref[...], preferred_element_type=_f32) + bp_ref[...]
    h_ref[...] = h
    a_ref[...] = jnp.dot(h, w1a_ref[...], preferred_element_type=_f32) + b1_ref[...]


def _prologue(x, Wp, bp, W1a, b1):
    return pl.pallas_call(
        _prologue_body,
        grid=(N // RN,),
        in_specs=[_rows((RN, D)), _full((D, D)), _full((1, D)),
                  _full((D, H)), _full((1, H))],
        out_specs=[_rows((RN, D)), _rows((RN, H))],
        out_shape=[jax.ShapeDtypeStruct((N, D), _f32),
                   jax.ShapeDtypeStruct((N_PAD, H), _f32)],
    )(x, Wp, bp, W1a, b1)


def _edge_mlp_body(g_ref, ea_ref, w1b_ref, w2_ref, b2_ref, m_ref):
    m1 = g_ref[...] + jnp.dot(ea_ref[...], w1b_ref[...],
                              preferred_element_type=_f32)
    m1 = jnp.maximum(m1, 0.0).astype(_bf16)
    m2 = jnp.dot(m1, w2_ref[...], preferred_element_type=_f32) + b2_ref[...]
    m_ref[...] = jnp.maximum(m2, 0.0)


def _edge_mlp(Gt, ea, W1b, W2, b2):
    return pl.pallas_call(
        _edge_mlp_body,
        grid=(E_PAD // RE,),
        in_specs=[_rows((RE, H)), _rows((RE, ED)), _full((ED, H)),
                  _full((H, D)), _full((1, D))],
        out_specs=_rows((RE, D)),
        out_shape=jax.ShapeDtypeStruct((E_PAD, D), _f32),
    )(Gt, ea, W1b, W2.astype(_bf16), b2)


def _ln_relu(z, g, be):
    mu = jnp.mean(z, axis=-1, keepdims=True)
    zc = z - mu
    var = jnp.mean(zc * zc, axis=-1, keepdims=True)
    y = zc * lax.rsqrt(var + 1e-5) * g + be
    return jnp.maximum(y, 0.0)


def _node_body(h_ref, pa_ref, pb_ref, g_ref, be_ref, wu_ref, bu_ref,
               w1a_ref, b1_ref, h1_ref, a1_ref):
    z = h_ref[...] + pa_ref[...] + pb_ref[...]
    r = _ln_relu(z, g_ref[...], be_ref[...])
    h1 = jnp.dot(r, wu_ref[...], preferred_element_type=_f32) + bu_ref[...]
    h1_ref[...] = h1
    a1_ref[...] = jnp.dot(h1, w1a_ref[...], preferred_element_type=_f32) + b1_ref[...]


def _node_update(h, Pa, Pb, g, be, Wu, bu, W1a, b1):
    return pl.pallas_call(
        _node_body,
        grid=(N // RN,),
        in_specs=[_rows((RN, D)), _rows((RN, D)), _rows((RN, D)),
                  _full((1, D)), _full((1, D)), _full((D, D)), _full((1, D)),
                  _full((D, H)), _full((1, H))],
        out_specs=[_rows((RN, D)), _rows((RN, H))],
        out_shape=[jax.ShapeDtypeStruct((N, D), _f32),
                   jax.ShapeDtypeStruct((N_PAD, H), _f32)],
    )(h, Pa, Pb, g, be, Wu, bu, W1a, b1)


def _final_body(h_ref, pa_ref, pb_ref, g_ref, be_ref, wu_ref, bu_ref,
                batch_ref, out_ref, acc_ref, cnt_ref):
    i = pl.program_id(0)

    @pl.when(i == 0)
    def _():
        acc_ref[...] = jnp.zeros_like(acc_ref)
        cnt_ref[...] = jnp.zeros_like(cnt_ref)

    z = h_ref[...] + pa_ref[...] + pb_ref[...]
    r = _ln_relu(z, g_ref[...], be_ref[...])
    h2 = jnp.dot(r, wu_ref[...], preferred_element_type=_f32) + bu_ref[...]

    b = batch_ref[...]
    gids = lax.broadcasted_iota(jnp.int32, (RN, G), 1)
    onehot = (b == gids).astype(_f32)
    dn = (((0,), (0,)), ((), ()))
    acc_ref[...] += lax.dot_general(onehot, h2, dn, preferred_element_type=_f32)
    cnt_ref[...] += lax.dot_general(onehot, jnp.ones((RN, D), _f32), dn,
                                    preferred_element_type=_f32)

    @pl.when(i == pl.num_programs(0) - 1)
    def _():
        out_ref[...] = acc_ref[...] / jnp.maximum(cnt_ref[...], 1.0)


def _final(h, Pa, Pb, g, be, Wu, bu, batch_col):
    return pl.pallas_call(
        _final_body,
        grid=(N // RN,),
        in_specs=[_rows((RN, D)), _rows((RN, D)), _rows((RN, D)),
                  _full((1, D)), _full((1, D)), _full((D, D)), _full((1, D)),
                  _rows((RN, 1))],
        out_specs=_full((G, D)),
        out_shape=jax.ShapeDtypeStruct((G, D), _f32),
        scratch_shapes=[pltpu.VMEM((G, D), _f32), pltpu.VMEM((G, D), _f32)],
    )(h, Pa, Pb, g, be, Wu, bu, batch_col)


NBG = 2
NBS = 2


@functools.cache
def _sc_kernels():
    mesh = plsc.VectorSubcoreMesh(core_axis_name="c", subcore_axis_name="s")

    @functools.partial(
        pl.kernel,
        out_type=jax.ShapeDtypeStruct((E_PAD, H), _f32),
        mesh=mesh,
        scratch_types=[pltpu.VMEM((CH, CHUNK), jnp.int32),
                       pltpu.VMEM((NBG, CHUNK, H), _f32),
                       pltpu.VMEM_SHARED((N_PAD, H), _f32),
                       pltpu.SemaphoreType.DMA((NBG,)),
                       pltpu.SemaphoreType.DMA((NBG,))],
    )
    def sc_gather(a_hbm, src_hbm, g_hbm, idx_v, rows_v, a_sh, gsem, wsem):
        c = lax.axis_index("c")
        s = lax.axis_index("s")
        wid = s * NC + c
        base = wid * CH * CHUNK
        tslice = pl.ds(s * ROWS_PER_TILE, ROWS_PER_TILE)
        pltpu.sync_copy(a_hbm.at[tslice], a_sh.at[tslice])
        pltpu.sync_copy(src_hbm.at[wid], idx_v)
        plsc.subcore_barrier()
        for b in range(NBG):
            pltpu.async_copy(a_sh.at[idx_v.at[b]], rows_v.at[b], gsem.at[b])

        @pl.loop(0, CH, step=NBG)
        def _(jo):
            for b in range(NBG):
                j = jo + b
                dst = g_hbm.at[pl.ds(base + j * CHUNK, CHUNK)]
                pltpu.make_async_copy(a_sh.at[idx_v.at[j]], rows_v.at[b],
                                      gsem.at[b]).wait()
                pltpu.async_copy(rows_v.at[b], dst, wsem.at[b])

                @pl.when(j + NBG < CH)
                def _():
                    pltpu.make_async_copy(rows_v.at[b], dst, wsem.at[b]).wait()
                    pltpu.async_copy(a_sh.at[idx_v.at[j + NBG]], rows_v.at[b],
                                     gsem.at[b])

        for b in range(NBG):
            pltpu.make_async_copy(
                rows_v.at[b], g_hbm.at[pl.ds(base, CHUNK)], wsem.at[b]).wait()

    @functools.partial(
        pl.kernel,
        out_type=jax.ShapeDtypeStruct((NC, N_PAD, D), _f32),
        mesh=mesh,
        scratch_types=[pltpu.VMEM((CH, CHUNK), jnp.int32),
                       pltpu.VMEM((NBS, CHUNK, D), _f32),
                       pltpu.VMEM_SHARED((N_PAD, D), _f32),
                       pltpu.SemaphoreType.DMA((NBS,))],
    )
    def sc_scatter(m_hbm, dst_hbm, z_hbm, p_hbm, idx_v, rows_v, agg, lsem):
        c = lax.axis_index("c")
        s = lax.axis_index("s")
        wid = s * NC + c
        base = wid * CH * CHUNK
        rslice = pl.ds(s * ROWS_PER_TILE, ROWS_PER_TILE)
        pltpu.sync_copy(z_hbm.at[rslice], agg.at[rslice])
        pltpu.sync_copy(dst_hbm.at[wid], idx_v)
        plsc.subcore_barrier()
        for b in range(NBS):
            pltpu.async_copy(m_hbm.at[pl.ds(base + b * CHUNK, CHUNK)],
                             rows_v.at[b], lsem.at[b])

        @pl.loop(0, CH, step=NBS)
        def _(jo):
            for b in range(NBS):
                j = jo + b
                pltpu.make_async_copy(m_hbm.at[pl.ds(base, CHUNK)],
                                      rows_v.at[b], lsem.at[b]).wait()
                pltpu.sync_copy(rows_v.at[b], agg.at[idx_v.at[j]], add=True)

                @pl.when(j + NBS < CH)
                def _():
                    pltpu.async_copy(
                        m_hbm.at[pl.ds(base + (j + NBS) * CHUNK, CHUNK)],
                        rows_v.at[b], lsem.at[b])

        plsc.subcore_barrier()
        pltpu.sync_copy(agg.at[rslice], p_hbm.at[c, rslice])

    return sc_gather, sc_scatter


def _sc_gather(a, src3):
    return _sc_kernels()[0](a, src3)


def _sc_scatter(m, dst3, zeros):
    return _sc_kernels()[1](m, dst3, zeros)


def kernel(x, edge_index, edge_attr, batch, Wp, bp,
           W1_0, b1_0, W2_0, b2_0, g_0, be_0, Wu_0, bu_0,
           W1_1, b1_1, W2_1, b2_1, g_1, be_1, Wu_1, bu_1):
    src = edge_index[0].astype(jnp.int32)
    dst = edge_index[1].astype(jnp.int32)
    src3 = jnp.reshape(jnp.pad(src, (0, E_PAD - E)), (NW, CH, CHUNK))
    dst3 = jnp.reshape(jnp.pad(dst, (0, E_PAD - E), constant_values=N),
                       (NW, CH, CHUNK))
    ea = jnp.pad(edge_attr, ((0, E_PAD - E), (0, 0)))
    zeros = jnp.zeros((N_PAD, D), _f32)
    batch_col = batch.astype(jnp.int32).reshape(N, 1)

    r2 = lambda v: v.reshape(1, -1)
    W1a_0, W1b_0 = W1_0[:D], W1_0[D:]
    W1a_1, W1b_1 = W1_1[:D], W1_1[D:]

    h0, A0 = _prologue(x, Wp, r2(bp), W1a_0, r2(b1_0))
    G0 = _sc_gather(A0, src3)
    M0 = _edge_mlp(G0, ea, W1b_0, W2_0, r2(b2_0))
    P0 = _sc_scatter(M0, dst3, zeros)
    h1, A1 = _node_update(h0, P0[0], P0[1], r2(g_0), r2(be_0), Wu_0, r2(bu_0),
                          W1a_1, r2(b1_1))
    G1 = _sc_gather(A1, src3)
    M1 = _edge_mlp(G1, ea, W1b_1, W2_1, r2(b2_1))
    P1 = _sc_scatter(M1, dst3, zeros)
    return _final(h1, P1[0], P1[1], r2(g_1), r2(be_1), Wu_1, r2(bu_1), batch_col)


# speedup vs baseline: 1.0103x; 1.0103x over previous
import functools

import jax
import jax.numpy as jnp
from jax import lax
from jax.experimental import pallas as pl
from jax.experimental.pallas import tpu as pltpu
from jax.experimental.pallas import tpu_sc as plsc

N = 10000
E = 320000
D = 128
ED = 16
H = 128
G = 64

NC = 2
NS = 16
NW = NC * NS
CHUNK = 128
CH = 80
CHH = CH // 2
E_HALF = NW * CHH * CHUNK
E_PAD = NW * CH * CHUNK
N_PAD = 10240
ROWS_PER_TILE = N_PAD // NS

RN = 2000
RE = 4096

_f32 = jnp.float32
_bf16 = jnp.bfloat16


def _full(shape):
    return pl.BlockSpec(shape, lambda i: (0,) * len(shape))


def _rows(shape):
    return pl.BlockSpec(shape, lambda i: (i,) + (0,) * (len(shape) - 1))


def _prologue_body(x_ref, wp_ref, bp_ref, w1a_ref, b1_ref, h_ref, a_ref):
    h = jnp.dot(x_ref[...], wp_ref[...], preferred_element_type=_f32) + bp_ref[...]
    h_ref[...] = h
    a_ref[...] = jnp.dot(h, w1a_ref[...], preferred_element_type=_f32) + b1_ref[...]


def _prologue(x, Wp, bp, W1a, b1):
    return pl.pallas_call(
        _prologue_body,
        grid=(N // RN,),
        in_specs=[_rows((RN, D)), _full((D, D)), _full((1, D)),
                  _full((D, H)), _full((1, H))],
        out_specs=[_rows((RN, D)), _rows((RN, H))],
        out_shape=[jax.ShapeDtypeStruct((N, D), _f32),
                   jax.ShapeDtypeStruct((N_PAD, H), _f32)],
    )(x, Wp, bp, W1a, b1)


def _edge_mlp_body(g_ref, ea_ref, w1b_ref, w2_ref, b2_ref, m_ref):
    m1 = g_ref[...] + jnp.dot(ea_ref[...], w1b_ref[...],
                              preferred_element_type=_f32)
    m1 = jnp.maximum(m1, 0.0).astype(_bf16)
    m2 = jnp.dot(m1, w2_ref[...], preferred_element_type=_f32) + b2_ref[...]
    m_ref[...] = jnp.maximum(m2, 0.0)


def _edge_mlp(Gt, ea, W1b, W2, b2):
    return pl.pallas_call(
        _edge_mlp_body,
        grid=(E_HALF // RE,),
        in_specs=[_rows((RE, H)), _rows((RE, ED)), _full((ED, H)),
                  _full((H, D)), _full((1, D))],
        out_specs=_rows((RE, D)),
        out_shape=jax.ShapeDtypeStruct((E_HALF, D), _f32),
    )(Gt, ea, W1b, W2.astype(_bf16), b2)


def _ln_relu(z, g, be):
    mu = jnp.mean(z, axis=-1, keepdims=True)
    zc = z - mu
    var = jnp.mean(zc * zc, axis=-1, keepdims=True)
    y = zc * lax.rsqrt(var + 1e-5) * g + be
    return jnp.maximum(y, 0.0)


def _node_body(h_ref, pa_ref, pb_ref, pc_ref, pd_ref, g_ref, be_ref, wu_ref,
               bu_ref, w1a_ref, b1_ref, h1_ref, a1_ref):
    z = (h_ref[...] + pa_ref[...] + pb_ref[...]) + (pc_ref[...] + pd_ref[...])
    r = _ln_relu(z, g_ref[...], be_ref[...])
    h1 = jnp.dot(r, wu_ref[...], preferred_element_type=_f32) + bu_ref[...]
    h1_ref[...] = h1
    a1_ref[...] = jnp.dot(h1, w1a_ref[...], preferred_element_type=_f32) + b1_ref[...]


def _node_update(h, Pa, Pb, Pc, Pd, g, be, Wu, bu, W1a, b1):
    return pl.pallas_call(
        _node_body,
        grid=(N // RN,),
        in_specs=[_rows((RN, D))] * 5 +
                 [_full((1, D)), _full((1, D)), _full((D, D)), _full((1, D)),
                  _full((D, H)), _full((1, H))],
        out_specs=[_rows((RN, D)), _rows((RN, H))],
        out_shape=[jax.ShapeDtypeStruct((N, D), _f32),
                   jax.ShapeDtypeStruct((N_PAD, H), _f32)],
    )(h, Pa, Pb, Pc, Pd, g, be, Wu, bu, W1a, b1)


def _final_body(h_ref, pa_ref, pb_ref, pc_ref, pd_ref, g_ref, be_ref, wu_ref,
                bu_ref, batch_ref, out_ref, acc_ref, cnt_ref):
    i = pl.program_id(0)

    @pl.when(i == 0)
    def _():
        acc_ref[...] = jnp.zeros_like(acc_ref)
        cnt_ref[...] = jnp.zeros_like(cnt_ref)

    z = (h_ref[...] + pa_ref[...] + pb_ref[...]) + (pc_ref[...] + pd_ref[...])
    r = _ln_relu(z, g_ref[...], be_ref[...])
    h2 = jnp.dot(r, wu_ref[...], preferred_element_type=_f32) + bu_ref[...]

    b = batch_ref[...]
    gids = lax.broadcasted_iota(jnp.int32, (RN, G), 1)
    onehot = (b == gids).astype(_f32)
    dn = (((0,), (0,)), ((), ()))
    acc_ref[...] += lax.dot_general(onehot, h2, dn, preferred_element_type=_f32)
    cnt_ref[...] += lax.dot_general(onehot, jnp.ones((RN, D), _f32), dn,
                                    preferred_element_type=_f32)

    @pl.when(i == pl.num_programs(0) - 1)
    def _():
        out_ref[...] = acc_ref[...] / jnp.maximum(cnt_ref[...], 1.0)


def _final(h, Pa, Pb, Pc, Pd, g, be, Wu, bu, batch_col):
    return pl.pallas_call(
        _final_body,
        grid=(N // RN,),
        in_specs=[_rows((RN, D))] * 5 +
                 [_full((1, D)), _full((1, D)), _full((D, D)), _full((1, D)),
                  _rows((RN, 1))],
        out_specs=_full((G, D)),
        out_shape=jax.ShapeDtypeStruct((G, D), _f32),
        scratch_shapes=[pltpu.VMEM((G, D), _f32), pltpu.VMEM((G, D), _f32)],
    )(h, Pa, Pb, Pc, Pd, g, be, Wu, bu, batch_col)


NBG = 2
NBS = 2


@functools.cache
def _sc_kernels():
    mesh = plsc.VectorSubcoreMesh(core_axis_name="c", subcore_axis_name="s")

    @functools.partial(
        pl.kernel,
        out_type=jax.ShapeDtypeStruct((E_HALF, H), _f32),
        mesh=mesh,
        scratch_types=[pltpu.VMEM((CHH, CHUNK), jnp.int32),
                       pltpu.VMEM((NBG, CHUNK, H), _f32),
                       pltpu.VMEM_SHARED((N_PAD, H), _f32),
                       pltpu.SemaphoreType.DMA((NBG,)),
                       pltpu.SemaphoreType.DMA((NBG,))],
    )
    def sc_gather(a_hbm, src_hbm, g_hbm, idx_v, rows_v, a_sh, gsem, wsem):
        c = lax.axis_index("c")
        s = lax.axis_index("s")
        wid = s * NC + c
        base = wid * CHH * CHUNK
        tslice = pl.ds(s * ROWS_PER_TILE, ROWS_PER_TILE)
        pltpu.sync_copy(a_hbm.at[tslice], a_sh.at[tslice])
        pltpu.sync_copy(src_hbm.at[wid], idx_v)
        plsc.subcore_barrier()
        for b in range(NBG):
            pltpu.async_copy(a_sh.at[idx_v.at[b]], rows_v.at[b], gsem.at[b])

        @pl.loop(0, CHH, step=NBG)
        def _(jo):
            for b in range(NBG):
                j = jo + b
                dst = g_hbm.at[pl.ds(base + j * CHUNK, CHUNK)]
                pltpu.make_async_copy(a_sh.at[idx_v.at[j]], rows_v.at[b],
                                      gsem.at[b]).wait()
                pltpu.async_copy(rows_v.at[b], dst, wsem.at[b])

                @pl.when(j + NBG < CHH)
                def _():
                    pltpu.make_async_copy(rows_v.at[b], dst, wsem.at[b]).wait()
                    pltpu.async_copy(a_sh.at[idx_v.at[j + NBG]], rows_v.at[b],
                                     gsem.at[b])

        for b in range(NBG):
            pltpu.make_async_copy(
                rows_v.at[b], g_hbm.at[pl.ds(base, CHUNK)], wsem.at[b]).wait()

    @functools.partial(
        pl.kernel,
        out_type=jax.ShapeDtypeStruct((NC, N_PAD, D), _f32),
        mesh=mesh,
        scratch_types=[pltpu.VMEM((CHH, CHUNK), jnp.int32),
                       pltpu.VMEM((NBS, CHUNK, D), _f32),
                       pltpu.VMEM_SHARED((N_PAD, D), _f32),
                       pltpu.SemaphoreType.DMA((NBS,))],
    )
    def sc_scatter(m_hbm, dst_hbm, z_hbm, p_hbm, idx_v, rows_v, agg, lsem):
        c = lax.axis_index("c")
        s = lax.axis_index("s")
        wid = s * NC + c
        base = wid * CHH * CHUNK
        rslice = pl.ds(s * ROWS_PER_TILE, ROWS_PER_TILE)
        pltpu.sync_copy(z_hbm.at[rslice], agg.at[rslice])
        pltpu.sync_copy(dst_hbm.at[wid], idx_v)
        plsc.subcore_barrier()
        for b in range(NBS):
            pltpu.async_copy(m_hbm.at[pl.ds(base + b * CHUNK, CHUNK)],
                             rows_v.at[b], lsem.at[b])

        @pl.loop(0, CHH, step=NBS)
        def _(jo):
            for b in range(NBS):
                j = jo + b
                pltpu.make_async_copy(m_hbm.at[pl.ds(base, CHUNK)],
                                      rows_v.at[b], lsem.at[b]).wait()
                pltpu.sync_copy(rows_v.at[b], agg.at[idx_v.at[j]], add=True)

                @pl.when(j + NBS < CHH)
                def _():
                    pltpu.async_copy(
                        m_hbm.at[pl.ds(base + (j + NBS) * CHUNK, CHUNK)],
                        rows_v.at[b], lsem.at[b])

        plsc.subcore_barrier()
        pltpu.sync_copy(agg.at[rslice], p_hbm.at[c, rslice])

    return sc_gather, sc_scatter


def _sc_gather(a, src3):
    return _sc_kernels()[0](a, src3)


def _sc_scatter(m, dst3, zeros):
    return _sc_kernels()[1](m, dst3, zeros)


def kernel(x, edge_index, edge_attr, batch, Wp, bp,
           W1_0, b1_0, W2_0, b2_0, g_0, be_0, Wu_0, bu_0,
           W1_1, b1_1, W2_1, b2_1, g_1, be_1, Wu_1, bu_1):
    src = edge_index[0].astype(jnp.int32)
    dst = edge_index[1].astype(jnp.int32)
    src_p = jnp.pad(src, (0, E_PAD - E))
    dst_p = jnp.pad(dst, (0, E_PAD - E), constant_values=N)
    src3 = [src_p[:E_HALF].reshape(NW, CHH, CHUNK),
            src_p[E_HALF:].reshape(NW, CHH, CHUNK)]
    dst3 = [dst_p[:E_HALF].reshape(NW, CHH, CHUNK),
            dst_p[E_HALF:].reshape(NW, CHH, CHUNK)]
    ea_p = jnp.pad(edge_attr, ((0, E_PAD - E), (0, 0)))
    ea = [ea_p[:E_HALF], ea_p[E_HALF:]]
    zeros = jnp.zeros((N_PAD, D), _f32)
    batch_col = batch.astype(jnp.int32).reshape(N, 1)

    r2 = lambda v: v.reshape(1, -1)
    W1a = [W1_0[:D], W1_1[:D]]
    W1b = [W1_0[D:], W1_1[D:]]
    W2 = [W2_0, W2_1]
    b1 = [r2(b1_0), r2(b1_1)]
    b2 = [r2(b2_0), r2(b2_1)]
    gg = [r2(g_0), r2(g_1)]
    be = [r2(be_0), r2(be_1)]
    Wu = [Wu_0, Wu_1]
    bu = [r2(bu_0), r2(bu_1)]

    h, A = _prologue(x, Wp, r2(bp), W1a[0], b1[0])
    for l in range(2):
        P = []
        for half in range(2):
            Gh = _sc_gather(A, src3[half])
            Mh = _edge_mlp(Gh, ea[half], W1b[l], W2[l], b2[l])
            P.append(_sc_scatter(Mh, dst3[half], zeros))
        parts = (P[0][0], P[0][1], P[1][0], P[1][1])
        if l == 0:
            h, A = _node_update(h, *parts, gg[0], be[0], Wu[0], bu[0],
                                W1a[1], b1[1])
        else:
            return _final(h, *parts, gg[1], be[1], Wu[1], bu[1], batch_col)


# speedup vs baseline: 1.0231x; 1.0127x over previous
import functools

import jax
import jax.numpy as jnp
from jax import lax
from jax.experimental import pallas as pl
from jax.experimental.pallas import tpu as pltpu
from jax.experimental.pallas import tpu_sc as plsc

N = 10000
E = 320000
D = 128
ED = 16
H = 128
G = 64

NC = 2
NS = 16
NW = NC * NS
CHUNK = 128
CH = 80
CHH = CH // 2
E_HALF = NW * CHH * CHUNK
E_PAD = NW * CH * CHUNK
N_PAD = 10240
ROWS_PER_TILE = N_PAD // NS

RN = 2000
RE = 8192

_f32 = jnp.float32
_bf16 = jnp.bfloat16


def _full(shape):
    return pl.BlockSpec(shape, lambda i: (0,) * len(shape))


def _rows(shape):
    return pl.BlockSpec(shape, lambda i: (i,) + (0,) * (len(shape) - 1))


def _prologue_body(x_ref, wp_ref, bp_ref, w1a_ref, b1_ref, h_ref, a_ref):
    h = jnp.dot(x_ref[...], wp_ref[...], preferred_element_type=_f32) + bp_ref[...]
    h_ref[...] = h
    a_ref[...] = jnp.dot(h, w1a_ref[...], preferred_element_type=_f32) + b1_ref[...]


def _prologue(x, Wp, bp, W1a, b1):
    return pl.pallas_call(
        _prologue_body,
        grid=(N // RN,),
        in_specs=[_rows((RN, D)), _full((D, D)), _full((1, D)),
                  _full((D, H)), _full((1, H))],
        out_specs=[_rows((RN, D)), _rows((RN, H))],
        out_shape=[jax.ShapeDtypeStruct((N, D), _f32),
                   jax.ShapeDtypeStruct((N_PAD, H), _f32)],
    )(x, Wp, bp, W1a, b1)


def _edge_mlp_body(g_ref, ea_ref, w1b_ref, w2_ref, b2_ref, m_ref):
    m1 = g_ref[...] + jnp.dot(ea_ref[...], w1b_ref[...],
                              preferred_element_type=_f32)
    m1 = jnp.maximum(m1, 0.0).astype(_bf16)
    m2 = jnp.dot(m1, w2_ref[...], preferred_element_type=_f32) + b2_ref[...]
    m_ref[...] = jnp.maximum(m2, 0.0)


def _edge_mlp(Gt, ea, W1b, W2, b2, off):
    return pl.pallas_call(
        _edge_mlp_body,
        grid=(E_HALF // RE,),
        in_specs=[pl.BlockSpec((RE, H), lambda i, o=off: (i + o, 0)),
                  _rows((RE, ED)), _full((ED, H)),
                  _full((H, D)), _full((1, D))],
        out_specs=_rows((RE, D)),
        out_shape=jax.ShapeDtypeStruct((E_HALF, D), _f32),
    )(Gt, ea, W1b, W2.astype(_bf16), b2)


def _ln_relu(z, g, be):
    mu = jnp.mean(z, axis=-1, keepdims=True)
    zc = z - mu
    var = jnp.mean(zc * zc, axis=-1, keepdims=True)
    y = zc * lax.rsqrt(var + 1e-5) * g + be
    return jnp.maximum(y, 0.0)


def _node_body(h_ref, pa_ref, pb_ref, pc_ref, pd_ref, g_ref, be_ref, wu_ref,
               bu_ref, w1a_ref, b1_ref, h1_ref, a1_ref):
    z = (h_ref[...] + pa_ref[...] + pb_ref[...]) + (pc_ref[...] + pd_ref[...])
    r = _ln_relu(z, g_ref[...], be_ref[...])
    h1 = jnp.dot(r, wu_ref[...], preferred_element_type=_f32) + bu_ref[...]
    h1_ref[...] = h1
    a1_ref[...] = jnp.dot(h1, w1a_ref[...], preferred_element_type=_f32) + b1_ref[...]


def _node_update(h, Pa, Pb, Pc, Pd, g, be, Wu, bu, W1a, b1):
    return pl.pallas_call(
        _node_body,
        grid=(N // RN,),
        in_specs=[_rows((RN, D))] * 5 +
                 [_full((1, D)), _full((1, D)), _full((D, D)), _full((1, D)),
                  _full((D, H)), _full((1, H))],
        out_specs=[_rows((RN, D)), _rows((RN, H))],
        out_shape=[jax.ShapeDtypeStruct((N, D), _f32),
                   jax.ShapeDtypeStruct((N_PAD, H), _f32)],
    )(h, Pa, Pb, Pc, Pd, g, be, Wu, bu, W1a, b1)


def _final_body(h_ref, pa_ref, pb_ref, pc_ref, pd_ref, g_ref, be_ref, wu_ref,
                bu_ref, batch_ref, out_ref, acc_ref, cnt_ref):
    i = pl.program_id(0)

    @pl.when(i == 0)
    def _():
        acc_ref[...] = jnp.zeros_like(acc_ref)
        cnt_ref[...] = jnp.zeros_like(cnt_ref)

    z = (h_ref[...] + pa_ref[...] + pb_ref[...]) + (pc_ref[...] + pd_ref[...])
    r = _ln_relu(z, g_ref[...], be_ref[...])
    h2 = jnp.dot(r, wu_ref[...], preferred_element_type=_f32) + bu_ref[...]

    b = batch_ref[...]
    gids = lax.broadcasted_iota(jnp.int32, (RN, G), 1)
    onehot = (b == gids).astype(_f32)
    dn = (((0,), (0,)), ((), ()))
    acc_ref[...] += lax.dot_general(onehot, h2, dn, preferred_element_type=_f32)
    cnt_ref[...] += lax.dot_general(onehot, jnp.ones((RN, D), _f32), dn,
                                    preferred_element_type=_f32)

    @pl.when(i == pl.num_programs(0) - 1)
    def _():
        out_ref[...] = acc_ref[...] / jnp.maximum(cnt_ref[...], 1.0)


def _final(h, Pa, Pb, Pc, Pd, g, be, Wu, bu, batch_col):
    return pl.pallas_call(
        _final_body,
        grid=(N // RN,),
        in_specs=[_rows((RN, D))] * 5 +
                 [_full((1, D)), _full((1, D)), _full((D, D)), _full((1, D)),
                  _rows((RN, 1))],
        out_specs=_full((G, D)),
        out_shape=jax.ShapeDtypeStruct((G, D), _f32),
        scratch_shapes=[pltpu.VMEM((G, D), _f32), pltpu.VMEM((G, D), _f32)],
    )(h, Pa, Pb, Pc, Pd, g, be, Wu, bu, batch_col)


NBG = 2
NBS = 2


@functools.cache
def _sc_kernels():
    mesh = plsc.VectorSubcoreMesh(core_axis_name="c", subcore_axis_name="s")

    @functools.partial(
        pl.kernel,
        out_type=jax.ShapeDtypeStruct((E_PAD, H), _f32),
        mesh=mesh,
        scratch_types=[pltpu.VMEM((CH, CHUNK), jnp.int32),
                       pltpu.VMEM((NBG, CHUNK, H), _f32),
                       pltpu.VMEM_SHARED((N_PAD, H), _f32),
                       pltpu.SemaphoreType.DMA((NBG,)),
                       pltpu.SemaphoreType.DMA((NBG,))],
    )
    def sc_gather(a_hbm, src_hbm, g_hbm, idx_v, rows_v, a_sh, gsem, wsem):
        c = lax.axis_index("c")
        s = lax.axis_index("s")
        wid = s * NC + c
        base = wid * CH * CHUNK
        tslice = pl.ds(s * ROWS_PER_TILE, ROWS_PER_TILE)
        pltpu.sync_copy(a_hbm.at[tslice], a_sh.at[tslice])
        pltpu.sync_copy(src_hbm.at[wid], idx_v)
        plsc.subcore_barrier()
        for b in range(NBG):
            pltpu.async_copy(a_sh.at[idx_v.at[b]], rows_v.at[b], gsem.at[b])

        @pl.loop(0, CH, step=NBG)
        def _(jo):
            for b in range(NBG):
                j = jo + b
                dst = g_hbm.at[pl.ds(base + j * CHUNK, CHUNK)]
                pltpu.make_async_copy(a_sh.at[idx_v.at[j]], rows_v.at[b],
                                      gsem.at[b]).wait()
                pltpu.async_copy(rows_v.at[b], dst, wsem.at[b])

                @pl.when(j + NBG < CH)
                def _():
                    pltpu.make_async_copy(rows_v.at[b], dst, wsem.at[b]).wait()
                    pltpu.async_copy(a_sh.at[idx_v.at[j + NBG]], rows_v.at[b],
                                     gsem.at[b])

        for b in range(NBG):
            pltpu.make_async_copy(
                rows_v.at[b], g_hbm.at[pl.ds(base, CHUNK)], wsem.at[b]).wait()

    @functools.partial(
        pl.kernel,
        out_type=jax.ShapeDtypeStruct((NC, N_PAD, D), _f32),
        mesh=mesh,
        scratch_types=[pltpu.VMEM((CHH, CHUNK), jnp.int32),
                       pltpu.VMEM((NBS, CHUNK, D), _f32),
                       pltpu.VMEM_SHARED((N_PAD, D), _f32),
                       pltpu.SemaphoreType.DMA((NBS,))],
    )
    def sc_scatter(m_hbm, dst_hbm, z_hbm, p_hbm, idx_v, rows_v, agg, lsem):
        c = lax.axis_index("c")
        s = lax.axis_index("s")
        wid = s * NC + c
        base = wid * CHH * CHUNK
        rslice = pl.ds(s * ROWS_PER_TILE, ROWS_PER_TILE)
        pltpu.sync_copy(z_hbm.at[rslice], agg.at[rslice])
        pltpu.sync_copy(dst_hbm.at[wid], idx_v)
        plsc.subcore_barrier()
        for b in range(NBS):
            pltpu.async_copy(m_hbm.at[pl.ds(base + b * CHUNK, CHUNK)],
                             rows_v.at[b], lsem.at[b])

        @pl.loop(0, CHH, step=NBS)
        def _(jo):
            for b in range(NBS):
                j = jo + b
                pltpu.make_async_copy(m_hbm.at[pl.ds(base, CHUNK)],
                                      rows_v.at[b], lsem.at[b]).wait()
                pltpu.sync_copy(rows_v.at[b], agg.at[idx_v.at[j]], add=True)

                @pl.when(j + NBS < CHH)
                def _():
                    pltpu.async_copy(
                        m_hbm.at[pl.ds(base + (j + NBS) * CHUNK, CHUNK)],
                        rows_v.at[b], lsem.at[b])

        plsc.subcore_barrier()
        pltpu.sync_copy(agg.at[rslice], p_hbm.at[c, rslice])

    return sc_gather, sc_scatter


def _sc_gather(a, src3):
    return _sc_kernels()[0](a, src3)


def _sc_scatter(m, dst3, zeros):
    return _sc_kernels()[1](m, dst3, zeros)


def kernel(x, edge_index, edge_attr, batch, Wp, bp,
           W1_0, b1_0, W2_0, b2_0, g_0, be_0, Wu_0, bu_0,
           W1_1, b1_1, W2_1, b2_1, g_1, be_1, Wu_1, bu_1):
    src = edge_index[0].astype(jnp.int32)
    dst = edge_index[1].astype(jnp.int32)
    src_p = jnp.pad(src, (0, E_PAD - E))
    dst_p = jnp.pad(dst, (0, E_PAD - E), constant_values=N)
    src3 = src_p.reshape(NW, CH, CHUNK)
    dst3 = [dst_p[:E_HALF].reshape(NW, CHH, CHUNK),
            dst_p[E_HALF:].reshape(NW, CHH, CHUNK)]
    ea_p = jnp.pad(edge_attr, ((0, E_PAD - E), (0, 0)))
    ea = [ea_p[:E_HALF], ea_p[E_HALF:]]
    zeros = jnp.zeros((N_PAD, D), _f32)
    batch_col = batch.astype(jnp.int32).reshape(N, 1)

    r2 = lambda v: v.reshape(1, -1)
    W1a = [W1_0[:D], W1_1[:D]]
    W1b = [W1_0[D:], W1_1[D:]]
    W2 = [W2_0, W2_1]
    b1 = [r2(b1_0), r2(b1_1)]
    b2 = [r2(b2_0), r2(b2_1)]
    gg = [r2(g_0), r2(g_1)]
    be = [r2(be_0), r2(be_1)]
    Wu = [Wu_0, Wu_1]
    bu = [r2(bu_0), r2(bu_1)]

    h, A = _prologue(x, Wp, r2(bp), W1a[0], b1[0])
    nblk = E_HALF // RE
    for l in range(2):
        Gf = _sc_gather(A, src3)
        P = []
        for half in range(2):
            Mh = _edge_mlp(Gf, ea[half], W1b[l], W2[l], b2[l], half * nblk)
            P.append(_sc_scatter(Mh, dst3[half], zeros))
        parts = (P[0][0], P[0][1], P[1][0], P[1][1])
        if l == 0:
            h, A = _node_update(h, *parts, gg[0], be[0], Wu[0], bu[0],
                                W1a[1], b1[1])
        else:
            return _final(h, *parts, gg[1], be[1], Wu[1], bu[1], batch_col)
